# Initial kernel scaffold; baseline (speedup 1.0000x reference)
import jax
import jax.numpy as jnp
from jax.experimental import pallas as pl


def kernel(bank, states, token_window, counts):
    raise NotImplementedError("write your pallas kernel here")


# speedup vs baseline: 4.6257x; 4.6257x over previous
import functools

import jax
import jax.numpy as jnp
import numpy as np
from jax import lax
from jax.experimental import pallas as pl
from jax.experimental.pallas import tpu as pltpu
from jax.experimental.pallas import tpu_sc as plsc

N_SLOTS = 500000
D = 16
NGRAM = 3
MOMENTUM = 0.9
B = 4096
L = 200
NT = B * L
ROWS128 = NT // 128

NC = 2
NS = 16
NW = NC * NS

_FNV_INIT = np.int64(2166136261).astype(np.int32)
_FNV_PRIME = np.int32(16777619)

BK = 50048
NP = 5


def _loop(n):
    def deco(body):
        def scan_body(i, _):
            body(i)
            return i + np.int32(1), None

        lax.scan(scan_body, np.int32(0), None, length=n)
    return deco


def _loop_c(n, init):
    def deco(body):
        def scan_body(carry, _):
            i, cur = carry
            cur = body(i, cur)
            return (i + np.int32(1), cur), None

        (_, cur), _ = lax.scan(scan_body, (np.int32(0), init), None, length=n)
        return cur
    return deco


def _hash_body(t0, t1, t2, pk_out):
    h = jnp.full(t0.shape, _FNV_INIT, dtype=jnp.int32)
    for t in (t0[...], t1[...], t2[...]):
        h = (h ^ t) * _FNV_PRIME
    hf = h.astype(jnp.float32) + jnp.where(
        h < 0, jnp.float32(4294967296.0), jnp.float32(0.0))
    q = (hf * jnp.float32(1.0 / N_SLOTS)).astype(jnp.int32)
    r = h - q * jnp.int32(N_SLOTS)
    r = r + jnp.where(r < 0, jnp.int32(N_SLOTS), jnp.int32(0))
    r = r - jnp.where(r >= N_SLOTS, jnp.int32(N_SLOTS), jnp.int32(0))
    qb = (r.astype(jnp.float32) * jnp.float32(1.0 / BK)).astype(jnp.int32)
    rem = r - qb * jnp.int32(BK)
    under = rem < 0
    qb = qb - jnp.where(under, jnp.int32(1), jnp.int32(0))
    rem = rem + jnp.where(under, jnp.int32(BK), jnp.int32(0))
    over = rem >= BK
    qb = qb + jnp.where(over, jnp.int32(1), jnp.int32(0))
    rem = rem - jnp.where(over, jnp.int32(BK), jnp.int32(0))
    pk_out[...] = qb * jnp.int32(65536) + rem


def _hash_addr(t0, t1, t2):
    return pl.pallas_call(
        _hash_body,
        out_shape=jax.ShapeDtypeStruct((ROWS128, 128), jnp.int32),
    )(t0, t1, t2)


_sc_mesh = plsc.VectorSubcoreMesh(core_axis_name="c", subcore_axis_name="s")

ACC = BK
DPAD = 2048
ACCP = ACC + DPAD
Z_ROWS = ACCP // NS
F_W = 512

G_CH = 2048
G_ROWS = G_CH // 128
TOK_W = NT // NW
G_NCH = -(-TOK_W // G_CH)

S_CH = 10240
S_NCH = (NT // NS) // S_CH
SGRP = 1024
NSUP = (S_CH + 128 + SGRP - 1) // SGRP
CMP_N = S_CH + 128

P_LEN = tuple(min(BK, N_SLOTS - q * BK) for q in range(2 * NP))


def _pw_split(plen):
    if plen % (NS * 8) == 0:
        pw_main = plen // NS
    else:
        pw_main = ((plen // NS + 7) // 8) * 8
    pw_last = plen - (NS - 1) * pw_main
    assert pw_main % 8 == 0 and pw_last % 8 == 0 and 0 < pw_last <= pw_main
    return pw_main, pw_last


@functools.partial(
    pl.kernel,
    out_type=jax.ShapeDtypeStruct((NT, D), jnp.float32),
    mesh=_sc_mesh,
    compiler_params=pltpu.CompilerParams(use_tc_tiling_on_sc=False,
                                         needs_layout_passes=False),
    scratch_types=[
        pltpu.VMEM((G_ROWS, 128), jnp.int32),
        pltpu.VMEM((G_CH, D), jnp.float32),
        pltpu.SemaphoreType.DMA,
    ],
)
def _gather_read(packed2d, bank, readout, aidx2d, grows_v, sem):
    c = lax.axis_index("c")
    s = lax.axis_index("s")
    wid = s * np.int32(NC) + c
    gt0 = wid * np.int32(TOK_W)

    @_loop(G_NCH)
    def _gchunk(ch):
        tb = gt0 + jnp.minimum(ch * np.int32(G_CH), np.int32(TOK_W - G_CH))
        row0 = pl.multiple_of(tb // np.int32(128), 8)
        pltpu.sync_copy(packed2d.at[pl.ds(row0, G_ROWS)], aidx2d)

        @_loop(G_CH // 16)
        def _unpk(i):
            rr = i >> np.int32(3)
            cc = (i & np.int32(7)) * np.int32(16)
            v = aidx2d[rr, pl.ds(cc, 16)]
            aidx2d[rr, pl.ds(cc, 16)] = \
                (v >> np.int32(16)) * np.int32(BK) + (v & np.int32(65535))
        cps = []
        for r in range(G_ROWS):
            cps.append(pltpu.async_copy(
                bank.at[aidx2d.at[np.int32(r)]],
                grows_v.at[pl.ds(r * 128, 128)], sem))
        for cp in cps:
            cp.wait()
        pltpu.sync_copy(grows_v,
                        readout.at[pl.ds(pl.multiple_of(tb, 8), G_CH)])


@functools.partial(
    pl.kernel,
    out_type=jax.ShapeDtypeStruct((N_SLOTS, D), jnp.float32),
    mesh=_sc_mesh,
    compiler_params=pltpu.CompilerParams(use_tc_tiling_on_sc=False,
                                         needs_layout_passes=False),
    scratch_types=[
        pltpu.VMEM((S_CH // 128, 128), jnp.int32),
        pltpu.VMEM((CMP_N,), jnp.int32),
        pltpu.VMEM((SGRP // 128, 128), jnp.int32),
        pltpu.VMEM((SGRP // 128, 128), jnp.int32),
        pltpu.VMEM((SGRP, D), jnp.float32),
        pltpu.VMEM((128,), jnp.float32),
        pltpu.VMEM((F_W, D), jnp.float32),
        pltpu.VMEM((2048,), jnp.float32),
        pltpu.VMEM((F_W, D), jnp.float32),
        pltpu.VMEM((F_W, D), jnp.float32),
        pltpu.VMEM((F_W,), jnp.float32),
        pltpu.VMEM((F_W,), jnp.int32),
        pltpu.VMEM((F_W, D), jnp.float32),
        pltpu.VMEM_SHARED((ACCP, D), jnp.float32),
        pltpu.VMEM_SHARED((ACCP,), jnp.float32),
        pltpu.SemaphoreType.DMA,
        pltpu.SemaphoreType.DMA,
    ],
)
def _scatter_ema(packed2d, states, bank, counts32, newbank,
                 pk2d, cmpbuf, gi2d, li2d, rows_v, ones_v,
                 zeros2d, zeros1d, bank_v, sums_v, cnt_v, c0_v, out_v,
                 sums_sh, cnts_sh, sem, sem2):
    c = lax.axis_index("c")
    s = lax.axis_index("s")
    lane = lax.iota(jnp.int32, 16)

    @_loop(8)
    def _fill(i):
        ones_v[pl.ds(i * np.int32(16), 16)] = jnp.full((16,), 1.0, jnp.float32)

    @_loop(128)
    def _fill1(i):
        zeros1d[pl.ds(i * np.int32(16), 16)] = jnp.full((16,), 0.0,
                                                        jnp.float32)

    @_loop(F_W)
    def _fillz(i):
        zeros2d[i] = jnp.full((D,), 0.0, jnp.float32)

    ts = s * np.int32(NT // NS)

    for p in range(NP):
        target = np.int32(2 * p) + c
        base = target * np.int32(BK)

        zb = s * np.int32(Z_ROWS)
        for w in range(Z_ROWS // F_W + 1):
            rb = min(w * F_W, Z_ROWS - F_W)
            pltpu.sync_copy(zeros2d, sums_sh.at[pl.ds(zb + np.int32(rb), F_W)])
        for w in range(Z_ROWS // 2048 + 1):
            rb = min(w * 2048, Z_ROWS - 2048)
            pltpu.sync_copy(zeros1d, cnts_sh.at[pl.ds(zb + np.int32(rb),
                                                      2048)])
        plsc.subcore_barrier()

        @_loop(S_NCH)
        def _chunk(j):
            row0 = pl.multiple_of(ts // np.int32(128) + j * np.int32(S_CH // 128), 8)
            pltpu.sync_copy(packed2d.at[pl.ds(row0, S_CH // 128)], pk2d)

            def _scan(i, cur):
                rr = i >> np.int32(3)
                cc = (i & np.int32(7)) * np.int32(16)
                v = pk2d[rr, pl.ds(cc, 16)]
                m = (v >> np.int32(16)) == target
                w = ((v & np.int32(65535)) * np.int32(16384)) + \
                    (i * np.int32(16) + lane)
                plsc.store_compressed(cmpbuf.at[pl.ds(cur, 16)], w, mask=m)
                npop = plsc.all_reduce_population_count(m)
                return cur + npop[0]

            cnt = _loop_c(S_CH // 16, np.int32(0))(_scan)

            @_loop(8)
            def _pad(t):
                dmy = (np.int32(ACC) + t * np.int32(16) + lane) \
                    * np.int32(16384)
                cmpbuf[pl.ds(cnt + t * np.int32(16), 16)] = dmy

            cnt_pad = ((cnt + np.int32(127)) >> np.int32(7)) * np.int32(128)

            @_loop(NSUP)
            def _sup(sup):
                sbase = sup * np.int32(SGRP)

                @pl.when(sbase < cnt_pad)
                def _do_sup():
                    @_loop(SGRP // 16)
                    def _build(vv):
                        vpos = sbase + vv * np.int32(16)

                        @pl.when(vpos < cnt_pad)
                        def _b1():
                            w = cmpbuf[pl.ds(vpos, 16)]
                            li = w >> np.int32(14)
                            gi = ts + j * np.int32(S_CH) + \
                                (w & np.int32(16383))
                            rr2 = vv >> np.int32(3)
                            cc2 = (vv & np.int32(7)) * np.int32(16)
                            gi2d[rr2, pl.ds(cc2, 16)] = gi
                            li2d[rr2, pl.ds(cc2, 16)] = li

                    @_loop(SGRP // 128)
                    def _fire(g):
                        @pl.when(sbase + g * np.int32(128) < cnt_pad)
                        def _f1():
                            pltpu.async_copy(
                                states.at[gi2d.at[g]],
                                rows_v.at[pl.ds(g * np.int32(128), 128)],
                                sem)

                    @_loop(SGRP // 128)
                    def _drain(g):
                        @pl.when(sbase + g * np.int32(128) < cnt_pad)
                        def _d1():
                            pltpu.make_async_copy(
                                states.at[gi2d.at[g]],
                                rows_v.at[pl.ds(g * np.int32(128), 128)],
                                sem).wait()

                    @_loop(SGRP // 128)
                    def _fire2(g):
                        @pl.when(sbase + g * np.int32(128) < cnt_pad)
                        def _f2():
                            pltpu.async_copy(
                                rows_v.at[pl.ds(g * np.int32(128), 128)],
                                sums_sh.at[li2d.at[g]], sem2, add=True)
                            pltpu.async_copy(
                                ones_v, cnts_sh.at[li2d.at[g]],
                                sem2, add=True)

                    @_loop(SGRP // 128)
                    def _drain2(g):
                        @pl.when(sbase + g * np.int32(128) < cnt_pad)
                        def _d2():
                            pltpu.make_async_copy(
                                rows_v.at[pl.ds(g * np.int32(128), 128)],
                                sums_sh.at[li2d.at[g]], sem2).wait()
                            pltpu.make_async_copy(
                                ones_v, cnts_sh.at[li2d.at[g]],
                                sem2).wait()

        plsc.subcore_barrier()

        if p < NP - 1:
            pw_main, pw_last = _pw_split(BK)
            pwv = jnp.where(s == np.int32(NS - 1), np.int32(pw_last),
                            np.int32(pw_main))
            sb = s * np.int32(pw_main)
            nwin = -(-pw_main // F_W)
        else:
            pm0, pl0 = _pw_split(P_LEN[2 * p])
            pm1, pl1 = _pw_split(P_LEN[2 * p + 1])
            pwm = jnp.where(c == 0, np.int32(pm0), np.int32(pm1))
            pwl = jnp.where(c == 0, np.int32(pl0), np.int32(pl1))
            pwv = jnp.where(s == np.int32(NS - 1), pwl, pwm)
            sb = s * pwm
            nwin = max(-(-pm0 // F_W), -(-pm1 // F_W))
        gb0 = base + sb

        @_loop(nwin)
        def _fin(w):
            rb = jnp.minimum(w * np.int32(F_W), pwv - np.int32(F_W))
            g0 = pl.multiple_of(gb0 + rb, 8)
            l0 = pl.multiple_of(sb + rb, 8)
            pltpu.sync_copy(bank.at[pl.ds(g0, F_W)], bank_v)
            pltpu.sync_copy(sums_sh.at[pl.ds(l0, F_W)], sums_v)
            pltpu.sync_copy(cnts_sh.at[pl.ds(l0, F_W)], cnt_v)
            pltpu.sync_copy(counts32.at[pl.ds(g0, F_W)], c0_v)

            @_loop(F_W // 16)
            def _grp(g):
                gg = g * np.int32(16)
                cnt = cnt_v[pl.ds(gg, 16)]
                c0 = c0_v[pl.ds(gg, 16)]
                hit = cnt > 0.0
                alpha = jnp.where(c0 == 0, jnp.float32(0.0),
                                  jnp.float32(MOMENTUM))
                safe = jnp.where(hit, cnt, jnp.float32(1.0))
                cb = jnp.where(hit, alpha, jnp.float32(1.0))
                cm = jnp.where(hit, (jnp.float32(1.0) - alpha) / safe,
                               jnp.float32(0.0))
                for k in range(16):
                    i = gg + np.int32(k)
                    out_v[i] = bank_v[i] * cb[k] + sums_v[i] * cm[k]

            pltpu.sync_copy(out_v, newbank.at[pl.ds(g0, F_W)])

        plsc.subcore_barrier()


def kernel(bank, states, token_window, counts):
    tw = token_window.reshape(NT, NGRAM).astype(jnp.int32)
    t0 = tw[:, 0].reshape(ROWS128, 128)
    t1 = tw[:, 1].reshape(ROWS128, 128)
    t2 = tw[:, 2].reshape(ROWS128, 128)
    packed2d = _hash_addr(t0, t1, t2)
    states2d = states.reshape(NT, D).astype(jnp.float32)
    counts32 = counts.astype(jnp.int32)
    read_flat = _gather_read(packed2d, bank)
    new_bank = _scatter_ema(packed2d, states2d, bank, counts32)
    return read_flat.reshape(B, L, D), new_bank
